# baseline (speedup 1.0000x reference)
import jax
import jax.numpy as jnp
from jax.experimental import pallas as pl
from jax.experimental.pallas import tpu as pltpu
from jax.experimental.pallas import tpu_sc as plsc


def kernel(input_or_shape, pos_table):
    batch, seq_len = input_or_shape.shape
    max_pos, hidden = pos_table.shape
    dtype = pos_table.dtype

    mesh = plsc.VectorSubcoreMesh(core_axis_name="c", subcore_axis_name="s")
    n_workers = mesh.num_cores * mesh.num_subcores
    chunk = seq_len // n_workers

    @pl.kernel(
        out_type=jax.ShapeDtypeStruct((batch, seq_len, hidden), dtype),
        mesh=mesh,
        scratch_types=[pltpu.SemaphoreType.DMA] * batch,
    )
    def sc_broadcast(tab_hbm, out_hbm, *sems):
        c = jax.lax.axis_index("c")
        s = jax.lax.axis_index("s")
        w = c * mesh.num_subcores + s
        base = w * chunk
        copies = [
            pltpu.async_copy(
                tab_hbm.at[pl.ds(base, chunk), :],
                out_hbm.at[b, pl.ds(base, chunk), :],
                sems[b],
            )
            for b in range(batch)
        ]
        for cp in copies:
            cp.wait()

    return sc_broadcast(pos_table)


# speedup vs baseline: 13.1455x; 13.1455x over previous
import jax
import jax.numpy as jnp
from jax.experimental import pallas as pl
from jax.experimental.pallas import tpu as pltpu
from jax.experimental.pallas import tpu_sc as plsc


def kernel(input_or_shape, pos_table):
    batch, seq_len = input_or_shape.shape
    max_pos, hidden = pos_table.shape
    dtype = pos_table.dtype

    sub = hidden // 128
    tab = pos_table.reshape(max_pos * sub, 128)

    position_ids = jnp.clip(jnp.arange(seq_len, dtype=jnp.int32), 0, max_pos - 1)
    sub_idx = position_ids[:, None] * sub + jnp.arange(sub, dtype=jnp.int32)[None, :]
    flat_idx = jnp.broadcast_to(
        sub_idx.reshape(1, seq_len * sub), (batch, seq_len * sub)
    ).reshape(1, batch * seq_len * sub)
    n_idx = batch * seq_len * sub

    window = 128
    mesh = plsc.VectorSubcoreMesh(core_axis_name="c", subcore_axis_name="s")

    @pl.kernel(
        out_type=jax.ShapeDtypeStruct((n_idx, 128), dtype),
        mesh=mesh,
    )
    def sc_gather(tab_hbm, i_hbm, o_hbm):
        def body(i_vmem, o_vmem):
            pltpu.sync_copy(tab_hbm.at[i_vmem.at[0]], o_vmem)

        pltpu.emit_pipeline(
            body,
            grid=(n_idx // window,),
            in_specs=[pl.BlockSpec((1, window), lambda i: (0, i))],
            out_specs=[pl.BlockSpec((window, 128), lambda i: (i, 0))],
            core_axis_name=("c", "s"),
            dimension_semantics=(pltpu.PARALLEL,),
        )(i_hbm, o_hbm)

    out = sc_gather(tab, flat_idx)
    return out.reshape(batch, seq_len, hidden)


# speedup vs baseline: 79.5764x; 6.0535x over previous
import jax
import jax.numpy as jnp
from jax.experimental import pallas as pl


def kernel(input_or_shape, pos_table):
    batch, seq_len = input_or_shape.shape
    max_pos, hidden = pos_table.shape

    block_s = 1024

    def body(tab_ref, out_ref):
        out_ref[...] = jnp.broadcast_to(tab_ref[...][None], (batch,) + tab_ref.shape)

    return pl.pallas_call(
        body,
        grid=(seq_len // block_s,),
        in_specs=[pl.BlockSpec((block_s, hidden), lambda i: (i, 0))],
        out_specs=pl.BlockSpec((batch, block_s, hidden), lambda i: (0, i, 0)),
        out_shape=jax.ShapeDtypeStruct((batch, seq_len, hidden), pos_table.dtype),
    )(pos_table)
